# baseline (speedup 1.0000x reference)
import jax
import jax.numpy as jnp
from jax import lax
from jax.experimental import pallas as pl
from jax.experimental.pallas import tpu as pltpu
from jax.experimental.pallas import tpu_sc as plsc

B = 16
C = 145
H = 128
W = 128
HW = H * W
N = 2048
NW = 32
NDET = N // NW
NELEM = NDET * C
DGRP = 8
NGRP = NDET // DGRP
LANES = 16
NV = 10


def _sc_body(pm_hbm, cm_hbm, bid_hbm, hw_hbm, meta_hbm,
             out_params, out_conf, out_preds, out_reorg,
             idx1d, buf, bids_v, hw_v, base_v, cidx_v, conf_buf,
             meta_v, reorg_buf, preds_buf, sem, sem2):
    wid = lax.axis_index("s") * 2 + lax.axis_index("c")
    det0 = wid * NDET

    cp_b = pltpu.async_copy(bid_hbm.at[pl.ds(det0, NDET)], bids_v, sem2)
    cp_h = pltpu.async_copy(hw_hbm.at[pl.ds(det0, NDET)], hw_v, sem2)
    cp_m = pltpu.async_copy(meta_hbm, meta_v, sem2)
    cp_b.wait()
    cp_h.wait()
    cp_m.wait()

    zeros = lax.iota(jnp.int32, LANES) * 0

    for t in range(NDET // LANES):
        sl = pl.ds(t * LANES, LANES)
        b = bids_v[sl]
        hw = hw_v[sl]
        base_v[sl] = b * (C * HW) + hw
        cidx_v[sl] = b * HW + hw
        reorg_buf[sl] = plsc.load_gather(meta_v, [b])
        px = (hw & (W - 1)).astype(jnp.float32) * 4.0
        py = lax.shift_right_logical(hw, 7).astype(jnp.float32) * 4.0
        pos = lax.iota(jnp.int32, LANES) * 2 + t * 2 * LANES
        plsc.store_scatter(preds_buf, [pos], px)
        plsc.store_scatter(preds_buf, [pos + 1], py)

    iot = lax.iota(jnp.int32, LANES)
    cpos = [iot + (v * LANES if v < NV - 1 else C - LANES) for v in range(NV)]
    chw = [c * HW for c in cpos]

    def gen_fire(g, _):
        for k in range(DGRP):
            n = g * DGRP + k
            n_spl = zeros + n
            bse = plsc.load_gather(base_v, [n_spl])
            p145 = n_spl * C
            for v in range(NV):
                plsc.store_scatter(idx1d, [p145 + cpos[v]], bse + chw[v])
        pltpu.async_copy(
            pm_hbm.at[idx1d.at[pl.ds(pl.multiple_of(g * DGRP * C, 8),
                                     DGRP * C)]],
            buf.at[pl.ds(pl.multiple_of(g * DGRP * C, 8), DGRP * C)],
            sem)
        return 0

    lax.fori_loop(0, NGRP, gen_fire, 0)

    cp_c = pltpu.async_copy(cm_hbm.at[cidx_v], conf_buf, sem2)
    cp_p = pltpu.async_copy(preds_buf, out_preds.at[pl.ds(det0 * 2, NDET * 2)],
                            sem2)
    cp_r = pltpu.async_copy(reorg_buf, out_reorg.at[pl.ds(det0, NDET)], sem2)
    cp_c.wait()
    cp_co = pltpu.async_copy(conf_buf, out_conf.at[pl.ds(det0, NDET)], sem2)

    def drain(g, _):
        o = pl.multiple_of(g * DGRP * C, 8)
        pltpu.make_async_copy(
            pm_hbm.at[idx1d.at[pl.ds(o, DGRP * C)]],
            buf.at[pl.ds(o, DGRP * C)],
            sem).wait()
        pltpu.async_copy(buf.at[pl.ds(o, DGRP * C)],
                         out_params.at[pl.ds(det0 * C + o, DGRP * C)],
                         sem2)
        return 0

    lax.fori_loop(0, NGRP, drain, 0)

    def drain_out(g, _):
        o = pl.multiple_of(g * DGRP * C, 8)
        pltpu.make_async_copy(buf.at[pl.ds(o, DGRP * C)],
                              out_params.at[pl.ds(det0 * C + o, DGRP * C)],
                              sem2).wait()
        return 0

    lax.fori_loop(0, NGRP, drain_out, 0)
    cp_p.wait()
    cp_r.wait()
    cp_co.wait()


@jax.jit
def kernel(params_maps, center_map, batch_ids, flat_inds, meta_batch_ids):
    pm_flat = params_maps.reshape(-1)
    cm_flat = center_map.reshape(-1)

    mesh = plsc.VectorSubcoreMesh(core_axis_name="c", subcore_axis_name="s")
    run = pl.kernel(
        _sc_body,
        out_type=(
            jax.ShapeDtypeStruct((N * C,), jnp.float32),
            jax.ShapeDtypeStruct((N,), jnp.float32),
            jax.ShapeDtypeStruct((N * 2,), jnp.float32),
            jax.ShapeDtypeStruct((N,), jnp.int32),
        ),
        mesh=mesh,
        compiler_params=pltpu.CompilerParams(needs_layout_passes=False),
        scratch_types=[
            pltpu.VMEM((NELEM,), jnp.int32),
            pltpu.VMEM((NELEM,), jnp.float32),
            pltpu.VMEM((NDET,), jnp.int32),
            pltpu.VMEM((NDET,), jnp.int32),
            pltpu.VMEM((NDET,), jnp.int32),
            pltpu.VMEM((NDET,), jnp.int32),
            pltpu.VMEM((NDET,), jnp.float32),
            pltpu.VMEM((B,), jnp.int32),
            pltpu.VMEM((NDET,), jnp.int32),
            pltpu.VMEM((NDET * 2,), jnp.float32),
            pltpu.SemaphoreType.DMA,
            pltpu.SemaphoreType.DMA,
        ],
    )
    params_flat, conf, preds, reorg = run(
        pm_flat, cm_flat, batch_ids, flat_inds, meta_batch_ids)

    params_pred = params_flat.reshape(N, C)
    center_preds = preds.reshape(N, 2)
    center_confs = conf.reshape(N, 1)
    return params_pred, center_preds, center_confs, reorg
